# baseline (speedup 1.0000x reference)
import functools

import jax
import jax.numpy as jnp
from jax import lax
from jax.experimental import pallas as pl
from jax.experimental.pallas import tpu as pltpu
from jax.experimental.pallas import tpu_sc as plsc

D = 1024
D2 = D // 2
E = 8
DFF = 4096
N_TOK = 8192
BT = 1024
NB = N_TOK // BT
A = 2 * N_TOK
TB = 512
NT = A // TB
NW = NT + E - 1

_NEG = -3.0e38


def _gating_body(x_ref, wg_ref, bg_ref, i0_ref, i1_ref, r0_ref, r1_ref,
                 w0_ref, w1_ref, hist_ref, cnt_ref, tri_ref):
    b = pl.program_id(0)

    @pl.when(b == 0)
    def _():
        cnt_ref[...] = jnp.zeros((E, 128), jnp.float32)
        ti = lax.broadcasted_iota(jnp.int32, (BT, BT), 0)
        tj = lax.broadcasted_iota(jnp.int32, (BT, BT), 1)
        tri_ref[...] = (ti < tj).astype(jnp.float32)

    lT = jax.lax.dot_general(
        wg_ref[...], x_ref[...], (((1,), (1,)), ((), ())),
        preferred_element_type=jnp.float32) + bg_ref[...].reshape(E, 1)

    eidx = lax.broadcasted_iota(jnp.int32, (E, BT), 0)
    m1 = jnp.max(lT, axis=0, keepdims=True)
    i1 = jnp.min(jnp.where(lT == m1, eidx, E), axis=0, keepdims=True)
    l2 = jnp.where(eidx == i1, _NEG, lT)
    m2 = jnp.max(l2, axis=0, keepdims=True)
    i2 = jnp.min(jnp.where(l2 == m2, eidx, E), axis=0, keepdims=True)

    w0 = 1.0 / (1.0 + jnp.exp(m2 - m1))
    w1 = 1.0 - w0

    o0 = (eidx == i1).astype(jnp.float32)
    o1 = (eidx == i2).astype(jnp.float32)
    osum = o0 + o1
    s = jax.lax.dot_general(osum, tri_ref[...], (((1,), (0,)), ((), ())),
                            preferred_element_type=jnp.float32)
    cnt = cnt_ref[...][:, :1]
    r0 = jnp.sum(o0 * (s + cnt), axis=0, keepdims=True)
    r1 = jnp.sum(o1 * (s + o0 + cnt), axis=0, keepdims=True)
    newc = cnt + jnp.sum(osum, axis=1, keepdims=True)
    cnt_ref[...] = jnp.broadcast_to(newc, (E, 128))

    i0_ref[...] = i1.reshape(1, 1, BT)
    i1_ref[...] = i2.reshape(1, 1, BT)
    r0_ref[...] = r0.astype(jnp.int32).reshape(1, 1, BT)
    r1_ref[...] = r1.astype(jnp.int32).reshape(1, 1, BT)
    w0_ref[...] = w0.reshape(1, 1, BT)
    w1_ref[...] = w1.reshape(1, 1, BT)
    hist_ref[...] = jnp.concatenate(
        [newc.reshape(1, E).astype(jnp.int32),
         jnp.zeros((1, 16 - E), jnp.int32)], axis=1)


def _gating_call(xf, Wg, bg):
    outs = [
        jax.ShapeDtypeStruct((NB, 1, BT), jnp.int32),
        jax.ShapeDtypeStruct((NB, 1, BT), jnp.int32),
        jax.ShapeDtypeStruct((NB, 1, BT), jnp.int32),
        jax.ShapeDtypeStruct((NB, 1, BT), jnp.int32),
        jax.ShapeDtypeStruct((NB, 1, BT), jnp.float32),
        jax.ShapeDtypeStruct((NB, 1, BT), jnp.float32),
        jax.ShapeDtypeStruct((1, 16), jnp.int32),
    ]
    blk = [pl.BlockSpec((1, 1, BT), lambda b: (b, 0, 0))] * 6 + [
        pl.BlockSpec((1, 16), lambda b: (0, 0))]
    return pl.pallas_call(
        _gating_body,
        grid=(NB,),
        in_specs=[
            pl.BlockSpec((BT, D), lambda b: (b, 0)),
            pl.BlockSpec((E, D), lambda b: (0, 0)),
            pl.BlockSpec((E,), lambda b: (0,)),
        ],
        out_specs=blk,
        out_shape=outs,
        scratch_shapes=[pltpu.VMEM((E, 128), jnp.float32),
                        pltpu.VMEM((BT, BT), jnp.float32)],
    )(xf, Wg, bg)


def _sc_mesh():
    return plsc.VectorSubcoreMesh(core_axis_name="c", subcore_axis_name="s")


_NTILES = 32
_CH = 32
_NCH_D = 8
_TPT = N_TOK // _NTILES


def _dispatch_body(x_hbm, i0_hbm, i1_hbm, r0_hbm, r1_hbm, offs_hbm,
                   xs_hbm, p0_hbm, p1_hbm,
                   obuf, ibuf, rbuf, p0buf, p1buf, xbuf,
                   sem0, sem1):
    wid = lax.axis_index("s") * 2 + lax.axis_index("c")
    tok0 = wid * _TPT

    pltpu.sync_copy(offs_hbm, obuf)

    waits = [None, None]
    for c in range(_NCH_D):
        b = c % 2
        sem = sem0 if b == 0 else sem1
        base = tok0 + c * _CH
        if waits[b] is not None:
            for h in waits[b]:
                h.wait()
            waits[b] = None
        for ibh, rbh, pbuf in ((i0_hbm, r0_hbm, p0buf), (i1_hbm, r1_hbm, p1buf)):
            pltpu.sync_copy(ibh.at[pl.ds(base, _CH)], ibuf)
            pltpu.sync_copy(rbh.at[pl.ds(base, _CH)], rbuf)
            for j in range(_CH // 16):
                e16 = ibuf[pl.ds(j * 16, 16)]
                r16 = rbuf[pl.ds(j * 16, 16)]
                o16 = plsc.load_gather(obuf, [e16])
                pbuf.at[b][pl.ds(j * 16, 16)] = r16 + o16
        pltpu.sync_copy(p0buf.at[b], p0_hbm.at[pl.ds(base, _CH)])
        pltpu.sync_copy(p1buf.at[b], p1_hbm.at[pl.ds(base, _CH)])
        pltpu.sync_copy(x_hbm.at[pl.ds(base, _CH)], xbuf.at[b])
        h0 = pltpu.async_copy(xbuf.at[b], xs_hbm.at[p0buf.at[b]], sem)
        h1 = pltpu.async_copy(xbuf.at[b], xs_hbm.at[p1buf.at[b]], sem)
        waits[b] = (h0, h1)
    for ws in waits:
        if ws is not None:
            for h in ws:
                h.wait()


def _dispatch_call(xf, i0, i1, r0, r1, offs):
    return pl.kernel(
        _dispatch_body,
        out_type=[
            jax.ShapeDtypeStruct((A, D), jnp.float32),
            jax.ShapeDtypeStruct((N_TOK,), jnp.int32),
            jax.ShapeDtypeStruct((N_TOK,), jnp.int32),
        ],
        mesh=_sc_mesh(),
        compiler_params=pltpu.CompilerParams(needs_layout_passes=False),
        scratch_types=[
            pltpu.VMEM((16,), jnp.int32),
            pltpu.VMEM((_CH,), jnp.int32),
            pltpu.VMEM((_CH,), jnp.int32),
            pltpu.VMEM((2, _CH), jnp.int32),
            pltpu.VMEM((2, _CH), jnp.int32),
            pltpu.VMEM((2, _CH, D), jnp.float32),
            pltpu.SemaphoreType.DMA,
            pltpu.SemaphoreType.DMA,
        ],
    )(xf, i0, i1, r0, r1, offs)


F2 = DFF // 2


def _ffn_body(t_ref, e_ref, lo_ref, hi_ref,
              xs_ref, w1_ref, b1_ref, w2_ref, b2_ref, out_ref, acc_ref):
    k = pl.program_id(0)
    f = pl.program_id(1)
    lo = lo_ref[k]
    hi = hi_ref[k]

    @pl.when(hi > lo)
    def _():
        xb = xs_ref[...]
        h = jax.lax.dot_general(xb, w1_ref[0, 0], (((1,), (1,)), ((), ())),
                                preferred_element_type=jnp.float32)
        h = jnp.maximum(h + b1_ref[0, 0, 0], 0.0).astype(jnp.bfloat16)
        y = jax.lax.dot_general(h, w2_ref[0], (((1,), (1,)), ((), ())),
                                preferred_element_type=jnp.float32)

        @pl.when(f == 0)
        def _():
            acc_ref[...] = y

        @pl.when(f == 1)
        def _():
            t = t_ref[k]
            row = t * TB + lax.broadcasted_iota(jnp.int32, (TB, 1), 0)
            valid = (row >= lo) & (row < hi)
            yout = acc_ref[...] + y + b2_ref[0, 0]
            out_ref[...] = jnp.where(valid, yout, out_ref[...])


def _ffn_call(wt, we, wlo, whi, xs, W1b, b1, W2b, b2):
    grid_spec = pltpu.PrefetchScalarGridSpec(
        num_scalar_prefetch=4,
        grid=(NW, 2),
        in_specs=[
            pl.BlockSpec((TB, D), lambda k, f, t, e, lo, hi: (t[k], 0)),
            pl.BlockSpec((1, 1, F2, D), lambda k, f, t, e, lo, hi: (e[k], f, 0, 0)),
            pl.BlockSpec((1, 1, 1, F2), lambda k, f, t, e, lo, hi: (e[k], f, 0, 0)),
            pl.BlockSpec((1, D, F2), lambda k, f, t, e, lo, hi: (e[k], 0, f)),
            pl.BlockSpec((1, 1, D), lambda k, f, t, e, lo, hi: (e[k], 0, 0)),
        ],
        out_specs=pl.BlockSpec((TB, D), lambda k, f, t, e, lo, hi: (t[k], 0)),
        scratch_shapes=[pltpu.VMEM((TB, D), jnp.float32)],
    )
    return pl.pallas_call(
        _ffn_body,
        grid_spec=grid_spec,
        out_shape=jax.ShapeDtypeStruct((A, D), jnp.float32),
    )(wt, we, wlo, whi, xs, W1b, b1, W2b, b2)


_CC = 16
_NCH_C = _TPT // _CC


def _combine_body(ys_hbm, p0_hbm, p1_hbm, w0_hbm, w1_hbm, out_hbm,
                  pbuf0, pbuf1, abuf, bbuf, obuf, wb0, wb1,
                  ga0, ga1, gb0, gb1, ws0, ws1):
    wid = lax.axis_index("s") * 2 + lax.axis_index("c")
    tok0 = wid * _TPT
    ga = (ga0, ga1)
    gb = (gb0, gb1)
    ws = (ws0, ws1)

    gwaits = [None, None]
    owaits = [None, None]
    for c in range(_NCH_C + 1):
        if c < _NCH_C:
            b = c % 2
            base = tok0 + c * _CC
            if owaits[b] is not None:
                owaits[b].wait()
                owaits[b] = None
            pltpu.sync_copy(p0_hbm.at[pl.ds(base, _CC)], pbuf0.at[b])
            pltpu.sync_copy(p1_hbm.at[pl.ds(base, _CC)], pbuf1.at[b])
            pltpu.sync_copy(w0_hbm.at[pl.ds(base, _CC)], wb0.at[b])
            pltpu.sync_copy(w1_hbm.at[pl.ds(base, _CC)], wb1.at[b])
            h0 = pltpu.async_copy(ys_hbm.at[pbuf0.at[b]], abuf.at[b], ga[b])
            h1 = pltpu.async_copy(ys_hbm.at[pbuf1.at[b]], bbuf.at[b], gb[b])
            gwaits[b] = (h0, h1)
        if c >= 1:
            bp = (c - 1) % 2
            base_p = tok0 + (c - 1) * _CC
            for h in gwaits[bp]:
                h.wait()

            def row(r, carry2):
                ridx = jnp.broadcast_to(r, (16,)).astype(jnp.int32)
                w0v = plsc.load_gather(wb0.at[bp], [ridx])
                w1v = plsc.load_gather(wb1.at[bp], [ridx])

                def vec(j, carry3):
                    av = abuf.at[bp][r, pl.ds(j * 16, 16)]
                    bv = bbuf.at[bp][r, pl.ds(j * 16, 16)]
                    obuf.at[bp][r, pl.ds(j * 16, 16)] = av * w0v + bv * w1v
                    return carry3

                return lax.fori_loop(0, D // 16, vec, carry2, unroll=4)

            lax.fori_loop(0, _CC, row, 0)
            owaits[bp] = pltpu.async_copy(
                obuf.at[bp], out_hbm.at[pl.ds(base_p, _CC)], ws[bp])
    for h in owaits:
        if h is not None:
            h.wait()


def _combine_call(ys, p0, p1, w0, w1):
    return pl.kernel(
        _combine_body,
        out_type=jax.ShapeDtypeStruct((N_TOK, D), jnp.float32),
        mesh=_sc_mesh(),
        compiler_params=pltpu.CompilerParams(needs_layout_passes=False),
        scratch_types=[
            pltpu.VMEM((2, _CC), jnp.int32),
            pltpu.VMEM((2, _CC), jnp.int32),
            pltpu.VMEM((2, _CC, D), jnp.float32),
            pltpu.VMEM((2, _CC, D), jnp.float32),
            pltpu.VMEM((2, _CC, D), jnp.float32),
            pltpu.VMEM((2, _CC), jnp.float32),
            pltpu.VMEM((2, _CC), jnp.float32),
            pltpu.SemaphoreType.DMA,
            pltpu.SemaphoreType.DMA,
            pltpu.SemaphoreType.DMA,
            pltpu.SemaphoreType.DMA,
            pltpu.SemaphoreType.DMA,
            pltpu.SemaphoreType.DMA,
        ],
    )(ys, p0, p1, w0, w1)


def _schedule(hist):
    off = jnp.concatenate(
        [jnp.zeros((1,), jnp.int32), jnp.cumsum(hist[0, :E], dtype=jnp.int32)])
    c = jnp.arange(NT * E, dtype=jnp.int32)
    t = c // E
    e = c % E
    lo = jnp.maximum(t * TB, off[e])
    hi = jnp.minimum((t + 1) * TB, off[e + 1])
    valid = hi > lo
    slot = jnp.where(valid, jnp.cumsum(valid.astype(jnp.int32)) - 1, NW)
    nvalid = jnp.sum(valid.astype(jnp.int32))
    wt = jnp.zeros((NW + 1,), jnp.int32).at[slot].set(t, mode="drop")
    we = jnp.zeros((NW + 1,), jnp.int32).at[slot].set(e, mode="drop")
    wlo = jnp.zeros((NW + 1,), jnp.int32).at[slot].set(lo, mode="drop")
    whi = jnp.zeros((NW + 1,), jnp.int32).at[slot].set(hi, mode="drop")
    pad = jnp.arange(NW) >= nvalid
    lt = wt[jnp.maximum(nvalid - 1, 0)]
    le = we[jnp.maximum(nvalid - 1, 0)]
    wt = jnp.where(pad, lt, wt[:NW])
    we = jnp.where(pad, le, we[:NW])
    wlo = jnp.where(pad, 0, wlo[:NW])
    whi = jnp.where(pad, 0, whi[:NW])
    offs16 = jnp.concatenate([off[:E], jnp.zeros((16 - E,), jnp.int32)])
    return wt, we, wlo, whi, offs16


def kernel(x, Wg, bg, W1, b1, W2, b2):
    xf = x.reshape(N_TOK, D)
    i0, i1, r0, r1, w0, w1, hist = _gating_call(xf, Wg, bg)
    wt, we, wlo, whi, offs16 = _schedule(hist)
    xs, p0, p1 = _dispatch_call(
        xf, i0.reshape(-1), i1.reshape(-1), r0.reshape(-1), r1.reshape(-1),
        offs16)
    ys = _ffn_call(wt, we, wlo, whi, xs.astype(jnp.bfloat16),
                   W1.astype(jnp.bfloat16).reshape(E, 2, F2, D),
                   b1.reshape(E, 2, 1, F2),
                   W2.astype(jnp.bfloat16), b2.reshape(E, 1, D))
    out = _combine_call(ys, p0, p1, w0.reshape(-1), w1.reshape(-1))
    return out.reshape(x.shape)


# speedup vs baseline: 1.0500x; 1.0500x over previous
import functools

import jax
import jax.numpy as jnp
from jax import lax
from jax.experimental import pallas as pl
from jax.experimental.pallas import tpu as pltpu
from jax.experimental.pallas import tpu_sc as plsc

D = 1024
D2 = D // 2
E = 8
DFF = 4096
N_TOK = 8192
BT = 1024
NB = N_TOK // BT
A = 2 * N_TOK
TB = 512
NT = A // TB
NW = NT + E - 1

_NEG = -3.0e38


def _gating_body(x_ref, wg_ref, bg_ref, i0_ref, i1_ref, r0_ref, r1_ref,
                 w0_ref, w1_ref, hist_ref, cnt_ref, tri_ref):
    b = pl.program_id(0)

    @pl.when(b == 0)
    def _():
        cnt_ref[...] = jnp.zeros((E, 128), jnp.float32)
        ti = lax.broadcasted_iota(jnp.int32, (BT, BT), 0)
        tj = lax.broadcasted_iota(jnp.int32, (BT, BT), 1)
        tri_ref[...] = (ti < tj).astype(jnp.float32)

    lT = jax.lax.dot_general(
        wg_ref[...], x_ref[...], (((1,), (1,)), ((), ())),
        preferred_element_type=jnp.float32) + bg_ref[...].reshape(E, 1)

    eidx = lax.broadcasted_iota(jnp.int32, (E, BT), 0)
    m1 = jnp.max(lT, axis=0, keepdims=True)
    i1 = jnp.min(jnp.where(lT == m1, eidx, E), axis=0, keepdims=True)
    l2 = jnp.where(eidx == i1, _NEG, lT)
    m2 = jnp.max(l2, axis=0, keepdims=True)
    i2 = jnp.min(jnp.where(l2 == m2, eidx, E), axis=0, keepdims=True)

    w0 = 1.0 / (1.0 + jnp.exp(m2 - m1))
    w1 = 1.0 - w0

    o0 = (eidx == i1).astype(jnp.float32)
    o1 = (eidx == i2).astype(jnp.float32)
    osum = o0 + o1
    s = jax.lax.dot_general(osum, tri_ref[...], (((1,), (0,)), ((), ())),
                            preferred_element_type=jnp.float32)
    cnt = cnt_ref[...][:, :1]
    r0 = jnp.sum(o0 * (s + cnt), axis=0, keepdims=True)
    r1 = jnp.sum(o1 * (s + o0 + cnt), axis=0, keepdims=True)
    newc = cnt + jnp.sum(osum, axis=1, keepdims=True)
    cnt_ref[...] = jnp.broadcast_to(newc, (E, 128))

    i0_ref[...] = i1.reshape(1, 1, BT)
    i1_ref[...] = i2.reshape(1, 1, BT)
    r0_ref[...] = r0.astype(jnp.int32).reshape(1, 1, BT)
    r1_ref[...] = r1.astype(jnp.int32).reshape(1, 1, BT)
    w0_ref[...] = w0.reshape(1, 1, BT)
    w1_ref[...] = w1.reshape(1, 1, BT)
    hist_ref[...] = jnp.concatenate(
        [newc.reshape(1, E).astype(jnp.int32),
         jnp.zeros((1, 16 - E), jnp.int32)], axis=1)


def _gating_call(xf, Wg, bg):
    outs = [
        jax.ShapeDtypeStruct((NB, 1, BT), jnp.int32),
        jax.ShapeDtypeStruct((NB, 1, BT), jnp.int32),
        jax.ShapeDtypeStruct((NB, 1, BT), jnp.int32),
        jax.ShapeDtypeStruct((NB, 1, BT), jnp.int32),
        jax.ShapeDtypeStruct((NB, 1, BT), jnp.float32),
        jax.ShapeDtypeStruct((NB, 1, BT), jnp.float32),
        jax.ShapeDtypeStruct((1, 16), jnp.int32),
    ]
    blk = [pl.BlockSpec((1, 1, BT), lambda b: (b, 0, 0))] * 6 + [
        pl.BlockSpec((1, 16), lambda b: (0, 0))]
    return pl.pallas_call(
        _gating_body,
        grid=(NB,),
        in_specs=[
            pl.BlockSpec((BT, D), lambda b: (b, 0)),
            pl.BlockSpec((E, D), lambda b: (0, 0)),
            pl.BlockSpec((E,), lambda b: (0,)),
        ],
        out_specs=blk,
        out_shape=outs,
        scratch_shapes=[pltpu.VMEM((E, 128), jnp.float32),
                        pltpu.VMEM((BT, BT), jnp.float32)],
    )(xf, Wg, bg)


def _sc_mesh():
    return plsc.VectorSubcoreMesh(core_axis_name="c", subcore_axis_name="s")


_NTILES = 32
_CH = 32
_NCH_D = 8
_TPT = N_TOK // _NTILES


def _dispatch_body(x_hbm, i0_hbm, i1_hbm, r0_hbm, r1_hbm, offs_hbm,
                   xs_hbm, p0_hbm, p1_hbm,
                   obuf, ibuf, rbuf, p0buf, p1buf, xbuf,
                   sem0, sem1):
    wid = lax.axis_index("s") * 2 + lax.axis_index("c")
    tok0 = wid * _TPT

    pltpu.sync_copy(offs_hbm, obuf)

    waits = [None, None]
    for c in range(_NCH_D):
        b = c % 2
        sem = sem0 if b == 0 else sem1
        base = tok0 + c * _CH
        if waits[b] is not None:
            for h in waits[b]:
                h.wait()
            waits[b] = None
        for ibh, rbh, pbuf in ((i0_hbm, r0_hbm, p0buf), (i1_hbm, r1_hbm, p1buf)):
            pltpu.sync_copy(ibh.at[pl.ds(base, _CH)], ibuf)
            pltpu.sync_copy(rbh.at[pl.ds(base, _CH)], rbuf)
            for j in range(_CH // 16):
                e16 = ibuf[pl.ds(j * 16, 16)]
                r16 = rbuf[pl.ds(j * 16, 16)]
                o16 = plsc.load_gather(obuf, [e16])
                pbuf.at[b][pl.ds(j * 16, 16)] = r16 + o16
        pltpu.sync_copy(p0buf.at[b], p0_hbm.at[pl.ds(base, _CH)])
        pltpu.sync_copy(p1buf.at[b], p1_hbm.at[pl.ds(base, _CH)])
        pltpu.sync_copy(x_hbm.at[pl.ds(base, _CH)], xbuf.at[b])
        h0 = pltpu.async_copy(xbuf.at[b], xs_hbm.at[p0buf.at[b]], sem)
        h1 = pltpu.async_copy(xbuf.at[b], xs_hbm.at[p1buf.at[b]], sem)
        waits[b] = (h0, h1)
    for ws in waits:
        if ws is not None:
            for h in ws:
                h.wait()


def _dispatch_call(xf, i0, i1, r0, r1, offs):
    return pl.kernel(
        _dispatch_body,
        out_type=[
            jax.ShapeDtypeStruct((A, D), jnp.float32),
            jax.ShapeDtypeStruct((N_TOK,), jnp.int32),
            jax.ShapeDtypeStruct((N_TOK,), jnp.int32),
        ],
        mesh=_sc_mesh(),
        compiler_params=pltpu.CompilerParams(needs_layout_passes=False),
        scratch_types=[
            pltpu.VMEM((16,), jnp.int32),
            pltpu.VMEM((_CH,), jnp.int32),
            pltpu.VMEM((_CH,), jnp.int32),
            pltpu.VMEM((2, _CH), jnp.int32),
            pltpu.VMEM((2, _CH), jnp.int32),
            pltpu.VMEM((2, _CH, D), jnp.float32),
            pltpu.SemaphoreType.DMA,
            pltpu.SemaphoreType.DMA,
        ],
    )(xf, i0, i1, r0, r1, offs)


def _ffn_body(t_ref, e_ref, lo_ref, hi_ref,
              xs_ref, w1_ref, b1_ref, w2_ref, b2_ref, out_ref):
    k = pl.program_id(0)
    lo = lo_ref[k]
    hi = hi_ref[k]

    @pl.when(hi > lo)
    def _():
        xb = xs_ref[...]
        h = jax.lax.dot_general(xb, w1_ref[0], (((1,), (1,)), ((), ())),
                                preferred_element_type=jnp.float32)
        h = jnp.maximum(h + b1_ref[0, 0], 0.0).astype(jnp.bfloat16)
        y = jax.lax.dot_general(h, w2_ref[0], (((1,), (1,)), ((), ())),
                                preferred_element_type=jnp.float32)
        t = t_ref[k]
        row = t * TB + lax.broadcasted_iota(jnp.int32, (TB, 1), 0)
        valid = (row >= lo) & (row < hi)
        out_ref[...] = jnp.where(valid, y + b2_ref[0, 0], out_ref[...])


def _ffn_call(wt, we, wlo, whi, xs, W1b, b1, W2b, b2):
    grid_spec = pltpu.PrefetchScalarGridSpec(
        num_scalar_prefetch=4,
        grid=(NW,),
        in_specs=[
            pl.BlockSpec((TB, D), lambda k, t, e, lo, hi: (t[k], 0)),
            pl.BlockSpec((1, DFF, D), lambda k, t, e, lo, hi: (e[k], 0, 0)),
            pl.BlockSpec((1, 1, DFF), lambda k, t, e, lo, hi: (e[k], 0, 0)),
            pl.BlockSpec((1, D, DFF), lambda k, t, e, lo, hi: (e[k], 0, 0)),
            pl.BlockSpec((1, 1, D), lambda k, t, e, lo, hi: (e[k], 0, 0)),
        ],
        out_specs=pl.BlockSpec((TB, D), lambda k, t, e, lo, hi: (t[k], 0)),
    )
    return pl.pallas_call(
        _ffn_body,
        grid_spec=grid_spec,
        out_shape=jax.ShapeDtypeStruct((A, D), jnp.float32),
    )(wt, we, wlo, whi, xs, W1b, b1, W2b, b2)


_CC = 16
_NCH_C = _TPT // _CC


def _combine_body(ys_hbm, p0_hbm, p1_hbm, w0_hbm, w1_hbm, out_hbm,
                  pbuf0, pbuf1, abuf, bbuf, obuf, wb0, wb1,
                  ga0, ga1, gb0, gb1, ws0, ws1):
    wid = lax.axis_index("s") * 2 + lax.axis_index("c")
    tok0 = wid * _TPT
    ga = (ga0, ga1)
    gb = (gb0, gb1)
    ws = (ws0, ws1)

    gwaits = [None, None]
    owaits = [None, None]
    for c in range(_NCH_C + 1):
        if c < _NCH_C:
            b = c % 2
            base = tok0 + c * _CC
            if owaits[b] is not None:
                owaits[b].wait()
                owaits[b] = None
            pltpu.sync_copy(p0_hbm.at[pl.ds(base, _CC)], pbuf0.at[b])
            pltpu.sync_copy(p1_hbm.at[pl.ds(base, _CC)], pbuf1.at[b])
            pltpu.sync_copy(w0_hbm.at[pl.ds(base, _CC)], wb0.at[b])
            pltpu.sync_copy(w1_hbm.at[pl.ds(base, _CC)], wb1.at[b])
            h0 = pltpu.async_copy(ys_hbm.at[pbuf0.at[b]], abuf.at[b], ga[b])
            h1 = pltpu.async_copy(ys_hbm.at[pbuf1.at[b]], bbuf.at[b], gb[b])
            gwaits[b] = (h0, h1)
        if c >= 1:
            bp = (c - 1) % 2
            base_p = tok0 + (c - 1) * _CC
            for h in gwaits[bp]:
                h.wait()

            def row(r, carry2):
                ridx = jnp.broadcast_to(r, (16,)).astype(jnp.int32)
                w0v = plsc.load_gather(wb0.at[bp], [ridx])
                w1v = plsc.load_gather(wb1.at[bp], [ridx])

                def vec(j, carry3):
                    av = abuf.at[bp][r, pl.ds(j * 16, 16)]
                    bv = bbuf.at[bp][r, pl.ds(j * 16, 16)]
                    obuf.at[bp][r, pl.ds(j * 16, 16)] = av * w0v + bv * w1v
                    return carry3

                return lax.fori_loop(0, D // 16, vec, carry2, unroll=4)

            lax.fori_loop(0, _CC, row, 0)
            owaits[bp] = pltpu.async_copy(
                obuf.at[bp], out_hbm.at[pl.ds(base_p, _CC)], ws[bp])
    for h in owaits:
        if h is not None:
            h.wait()


def _combine_call(ys, p0, p1, w0, w1):
    return pl.kernel(
        _combine_body,
        out_type=jax.ShapeDtypeStruct((N_TOK, D), jnp.float32),
        mesh=_sc_mesh(),
        compiler_params=pltpu.CompilerParams(needs_layout_passes=False),
        scratch_types=[
            pltpu.VMEM((2, _CC), jnp.int32),
            pltpu.VMEM((2, _CC), jnp.int32),
            pltpu.VMEM((2, _CC, D), jnp.float32),
            pltpu.VMEM((2, _CC, D), jnp.float32),
            pltpu.VMEM((2, _CC, D), jnp.float32),
            pltpu.VMEM((2, _CC), jnp.float32),
            pltpu.VMEM((2, _CC), jnp.float32),
            pltpu.SemaphoreType.DMA,
            pltpu.SemaphoreType.DMA,
            pltpu.SemaphoreType.DMA,
            pltpu.SemaphoreType.DMA,
            pltpu.SemaphoreType.DMA,
            pltpu.SemaphoreType.DMA,
        ],
    )(ys, p0, p1, w0, w1)


def _schedule(hist):
    off = jnp.concatenate(
        [jnp.zeros((1,), jnp.int32), jnp.cumsum(hist[0, :E], dtype=jnp.int32)])
    c = jnp.arange(NT * E, dtype=jnp.int32)
    t = c // E
    e = c % E
    lo = jnp.maximum(t * TB, off[e])
    hi = jnp.minimum((t + 1) * TB, off[e + 1])
    valid = hi > lo
    slot = jnp.where(valid, jnp.cumsum(valid.astype(jnp.int32)) - 1, NW)
    nvalid = jnp.sum(valid.astype(jnp.int32))
    wt = jnp.zeros((NW + 1,), jnp.int32).at[slot].set(t, mode="drop")
    we = jnp.zeros((NW + 1,), jnp.int32).at[slot].set(e, mode="drop")
    wlo = jnp.zeros((NW + 1,), jnp.int32).at[slot].set(lo, mode="drop")
    whi = jnp.zeros((NW + 1,), jnp.int32).at[slot].set(hi, mode="drop")
    pad = jnp.arange(NW) >= nvalid
    lt = wt[jnp.maximum(nvalid - 1, 0)]
    le = we[jnp.maximum(nvalid - 1, 0)]
    wt = jnp.where(pad, lt, wt[:NW])
    we = jnp.where(pad, le, we[:NW])
    wlo = jnp.where(pad, 0, wlo[:NW])
    whi = jnp.where(pad, 0, whi[:NW])
    offs16 = jnp.concatenate([off[:E], jnp.zeros((16 - E,), jnp.int32)])
    return wt, we, wlo, whi, offs16


def kernel(x, Wg, bg, W1, b1, W2, b2):
    xf = x.reshape(N_TOK, D)
    i0, i1, r0, r1, w0, w1, hist = _gating_call(xf, Wg, bg)
    wt, we, wlo, whi, offs16 = _schedule(hist)
    xs, p0, p1 = _dispatch_call(
        xf, i0.reshape(-1), i1.reshape(-1), r0.reshape(-1), r1.reshape(-1),
        offs16)
    ys = _ffn_call(wt, we, wlo, whi, xs.astype(jnp.bfloat16),
                   W1.astype(jnp.bfloat16), b1.reshape(E, 1, DFF),
                   W2.astype(jnp.bfloat16), b2.reshape(E, 1, D))
    out = _combine_call(ys, p0, p1, w0.reshape(-1), w1.reshape(-1))
    return out.reshape(x.shape)
